# baseline (speedup 1.0000x reference)
import functools

import jax
import jax.numpy as jnp
from jax import lax
from jax.experimental import pallas as pl
from jax.experimental.pallas import tpu as pltpu
from jax.experimental.pallas import tpu_sc as plsc

T = 4096
HIDDEN = 2048
INTER = 1024
E = 8
K = 2
TK = T * K

BM = 256
NB = TK // BM + E
P = NB * BM

NC = 2
NS = 16
NW = NC * NS

GCH = 16
CT = 8


@functools.lru_cache(maxsize=None)
def _sc_mesh():
    return plsc.VectorSubcoreMesh(
        core_axis_name="c", subcore_axis_name="s",
        num_cores=NC, num_subcores=NS)


def _wid():
    return lax.axis_index("s") * NC + lax.axis_index("c")


def _gather_body(tab_hbm, pa_hbm, pb_hbm, out_hbm,
                 sidxa0, sidxa1, sidxb0, sidxb1, rows0, rows1,
                 gsem0, gsem1, ssema0, ssema1, ssemb0, ssemb1):
    per_w = T // NW
    base = _wid() * per_w
    n_ch = per_w // GCH
    sidxa = (sidxa0, sidxa1)
    sidxb = (sidxb0, sidxb1)
    rows = (rows0, rows1)
    gsem = (gsem0, gsem1)
    ssema = (ssema0, ssema1)
    ssemb = (ssemb0, ssemb1)

    def start(c):
        b = c % 2
        lo = base + c * GCH
        pltpu.sync_copy(pa_hbm.at[pl.ds(lo, GCH)], sidxa[b])
        pltpu.sync_copy(pb_hbm.at[pl.ds(lo, GCH)], sidxb[b])
        return pltpu.async_copy(tab_hbm.at[pl.ds(lo, GCH)], rows[b], gsem[b])

    gd = {0: start(0)}
    sd = {}
    for c in range(n_ch):
        b = c % 2
        if c + 1 < n_ch:
            if c - 1 in sd:
                sd[c - 1][0].wait()
                sd[c - 1][1].wait()
            gd[c + 1] = start(c + 1)
        gd[c].wait()
        sd[c] = (pltpu.async_copy(rows[b], out_hbm.at[sidxa[b]], ssema[b]),
                 pltpu.async_copy(rows[b], out_hbm.at[sidxb[b]], ssemb[b]))
    for c in (n_ch - 2, n_ch - 1):
        if c >= 0 and c in sd:
            sd[c][0].wait()
            sd[c][1].wait()


@functools.lru_cache(maxsize=None)
def _gather_call():
    return pl.kernel(
        _gather_body,
        out_type=jax.ShapeDtypeStruct((P, HIDDEN), jnp.float32),
        mesh=_sc_mesh(),
        scratch_types=[
            pltpu.VMEM((GCH,), jnp.int32),
            pltpu.VMEM((GCH,), jnp.int32),
            pltpu.VMEM((GCH,), jnp.int32),
            pltpu.VMEM((GCH,), jnp.int32),
            pltpu.VMEM((GCH, HIDDEN), jnp.float32),
            pltpu.VMEM((GCH, HIDDEN), jnp.float32),
            pltpu.SemaphoreType.DMA,
            pltpu.SemaphoreType.DMA,
            pltpu.SemaphoreType.DMA,
            pltpu.SemaphoreType.DMA,
            pltpu.SemaphoreType.DMA,
            pltpu.SemaphoreType.DMA,
        ],
    )


def _ffn_body(be_ref, x_ref, w_ref, wgu_ref, wd_ref, o_ref):
    @pl.when(pl.program_id(0) < be_ref[NB])
    def _():
        x = x_ref[...].astype(jnp.bfloat16)
        gu = lax.dot_general(
            x, wgu_ref[0],
            (((1,), (1,)), ((), ())),
            preferred_element_type=jnp.float32)
        g = gu[:, :INTER]
        u = gu[:, INTER:]
        h = (g * lax.logistic(g) * u).astype(jnp.bfloat16)
        out = lax.dot_general(
            h, wd_ref[0],
            (((1,), (1,)), ((), ())),
            preferred_element_type=jnp.float32)
        o_ref[...] = out * w_ref[...]


def _grouped_ffn(bexp, xp, w_pos, wgu, wd):
    grid_spec = pltpu.PrefetchScalarGridSpec(
        num_scalar_prefetch=1,
        grid=(NB,),
        in_specs=[
            pl.BlockSpec((BM, HIDDEN), lambda b, be: (b, 0)),
            pl.BlockSpec((BM, 1), lambda b, be: (b, 0)),
            pl.BlockSpec((1, 2 * INTER, HIDDEN), lambda b, be: (be[b], 0, 0)),
            pl.BlockSpec((1, HIDDEN, INTER), lambda b, be: (be[b], 0, 0)),
        ],
        out_specs=pl.BlockSpec((BM, HIDDEN), lambda b, be: (b, 0)),
    )
    return pl.pallas_call(
        _ffn_body,
        grid_spec=grid_spec,
        out_shape=jax.ShapeDtypeStruct((P, HIDDEN), jnp.float32),
        compiler_params=pltpu.CompilerParams(
            dimension_semantics=("arbitrary",),
            vmem_limit_bytes=110 * 1024 * 1024,
        ),
    )(bexp, xp, w_pos, wgu, wd)


def _combine_body(po_hbm, pos_hbm, out_hbm,
                  ii0, ii1, bi0, bi1, bo0, bo1,
                  sg0, sg1, so0, so1):
    per_w = T // NW
    tbase = _wid() * per_w
    n_ch = per_w // CT
    ii = (ii0, ii1)
    bi = (bi0, bi1)
    bo = (bo0, bo1)
    sg = (sg0, sg1)
    so = (so0, so1)

    def start(c):
        b = c % 2
        pltpu.sync_copy(pos_hbm.at[pl.ds((tbase + c * CT) * K, CT * K)], ii[b])
        return pltpu.async_copy(po_hbm.at[ii[b]], bi[b], sg[b])

    gd = {0: start(0)}
    wd_ = {}
    for c in range(n_ch):
        b = c % 2
        if c + 1 < n_ch:
            if c - 1 in wd_:
                wd_[c - 1].wait()
            gd[c + 1] = start(c + 1)
        gd[c].wait()
        for r in range(CT):
            @plsc.parallel_loop(0, HIDDEN // 16, unroll=8)
            def dbody(d, r=r, b=b):
                s = pl.ds(d * 16, 16)
                bo[b][r, s] = bi[b][2 * r, s] + bi[b][2 * r + 1, s]
        wd_[c] = pltpu.async_copy(
            bo[b], out_hbm.at[pl.ds(tbase + c * CT, CT)], so[b])
    for c in (n_ch - 2, n_ch - 1):
        if c >= 0 and c in wd_:
            wd_[c].wait()


@functools.lru_cache(maxsize=None)
def _combine_call():
    return pl.kernel(
        _combine_body,
        out_type=jax.ShapeDtypeStruct((T, HIDDEN), jnp.float32),
        mesh=_sc_mesh(),
        scratch_types=[
            pltpu.VMEM((CT * K,), jnp.int32),
            pltpu.VMEM((CT * K,), jnp.int32),
            pltpu.VMEM((CT * K, HIDDEN), jnp.float32),
            pltpu.VMEM((CT * K, HIDDEN), jnp.float32),
            pltpu.VMEM((CT, HIDDEN), jnp.float32),
            pltpu.VMEM((CT, HIDDEN), jnp.float32),
            pltpu.SemaphoreType.DMA,
            pltpu.SemaphoreType.DMA,
            pltpu.SemaphoreType.DMA,
            pltpu.SemaphoreType.DMA,
        ],
    )


def kernel(hidden_states, top_k_index, top_k_weights, gate_up_proj, down_proj):
    idx = top_k_index.astype(jnp.int32).reshape(-1)
    oh = (idx[:, None] == jnp.arange(E, dtype=jnp.int32)).astype(jnp.int32)
    csum = jnp.cumsum(oh, axis=0)
    counts = csum[-1]
    rank = jnp.take_along_axis(csum, idx[:, None], axis=1)[:, 0] - 1
    nb = (counts + BM - 1) // BM
    bcum = jnp.cumsum(nb)
    off = (bcum - nb) * BM
    pos = (off[idx] + rank).astype(jnp.int32)
    bexp = jnp.minimum(
        jnp.searchsorted(bcum, jnp.arange(NB, dtype=jnp.int32), side="right"),
        E - 1,
    ).astype(jnp.int32)
    bexp_ext = jnp.concatenate([bexp, bcum[-1:].astype(jnp.int32)])
    w_pos = jnp.zeros((P,), jnp.float32).at[pos].set(
        top_k_weights.reshape(-1).astype(jnp.float32)).reshape(P, 1)

    pos2 = pos.reshape(T, K)
    packed_x = _gather_call()(hidden_states, pos2[:, 0], pos2[:, 1])

    packed_out = _grouped_ffn(bexp_ext, packed_x, w_pos,
        gate_up_proj.astype(jnp.bfloat16), down_proj.astype(jnp.bfloat16))

    final = _combine_call()(packed_out, pos)
    return final


# speedup vs baseline: 1.3804x; 1.3804x over previous
import functools

import jax
import jax.numpy as jnp
from jax import lax
from jax.experimental import pallas as pl
from jax.experimental.pallas import tpu as pltpu
from jax.experimental.pallas import tpu_sc as plsc

T = 4096
HIDDEN = 2048
INTER = 1024
E = 8
K = 2
TK = T * K

BM = 256
NB = TK // BM + E
P = NB * BM

NC = 2
NS = 16
NW = NC * NS

GCH = 16
CT = 8


@functools.lru_cache(maxsize=None)
def _sc_mesh():
    return plsc.VectorSubcoreMesh(
        core_axis_name="c", subcore_axis_name="s",
        num_cores=NC, num_subcores=NS)


def _wid():
    return lax.axis_index("s") * NC + lax.axis_index("c")


NBE = 48
MPW = TK // NS


def _meta1_body(tki_hbm, rank_hbm, cnts_hbm, idx_v, rank_v, cnt_stage):
    s = lax.axis_index("s")
    pltpu.sync_copy(tki_hbm.at[pl.ds(s * MPW, MPW)], idx_v)
    lanes = lax.iota(jnp.int32, 16)
    zeros = jnp.zeros((16,), jnp.int32)
    cnt_stage[0, :] = zeros
    for c in range(MPW // 16):
        v = idx_v[pl.ds(c * 16, 16)]
        base = plsc.load_gather(cnt_stage, [zeros, v])
        prefsel = zeros
        c_vec = cnt_stage[0, :]
        for e in range(E):
            m = v == e
            prefsel = jnp.where(m, plsc.cumsum(m.astype(jnp.int32)),
                                prefsel)
            pcnt = plsc.all_reduce_population_count(m)
            c_vec = c_vec + pcnt * (lanes == e).astype(jnp.int32)
        cnt_stage[0, :] = c_vec
        rank_v[pl.ds(c * 16, 16)] = base + prefsel - 1
    pltpu.sync_copy(rank_v, rank_hbm.at[pl.ds(s * MPW, MPW)])
    pltpu.sync_copy(cnt_stage, cnts_hbm.at[pl.ds(s, 1)])


@functools.lru_cache(maxsize=None)
def _meta1_call():
    return pl.kernel(
        _meta1_body,
        out_type=(jax.ShapeDtypeStruct((TK,), jnp.int32),
                  jax.ShapeDtypeStruct((NS, 16), jnp.int32)),
        mesh=_sc_mesh(),
        compiler_params=pltpu.CompilerParams(needs_layout_passes=False),
        scratch_types=[
            pltpu.VMEM((MPW,), jnp.int32),
            pltpu.VMEM((MPW,), jnp.int32),
            pltpu.VMEM((1, 16), jnp.int32),
        ],
    )


def _meta2_body(tki_hbm, rank_hbm, cnts_hbm, pos_hbm,
                idx_v, rank_v, cnts_all, base_ref, cumnb_ref):
    s = lax.axis_index("s")
    pltpu.sync_copy(tki_hbm.at[pl.ds(s * MPW, MPW)], idx_v)
    pltpu.sync_copy(rank_hbm.at[pl.ds(s * MPW, MPW)], rank_v)
    pltpu.sync_copy(cnts_hbm, cnts_all)
    zeros = jnp.zeros((16,), jnp.int32)
    s_vec = zeros + s
    tot = zeros
    prefm = zeros
    for wp in range(NS):
        row = cnts_all[wp, :]
        tot = tot + row
        prefm = prefm + row * (s_vec > wp).astype(jnp.int32)
    nbv = lax.shift_right_logical(tot + (BM - 1), 8)
    cumnb = plsc.cumsum(nbv)
    cumnb_ref[...] = cumnb
    blk_off = lax.shift_left(cumnb - nbv, 8)
    base_ref[...] = blk_off + prefm
    for c in range(MPW // 16):
        v = idx_v[pl.ds(c * 16, 16)]
        b = plsc.load_gather(base_ref, [v])
        rank_v[pl.ds(c * 16, 16)] = b + rank_v[pl.ds(c * 16, 16)]
    pltpu.sync_copy(rank_v, pos_hbm.at[pl.ds(s * MPW, MPW)])


@functools.lru_cache(maxsize=None)
def _meta2_call():
    return pl.kernel(
        _meta2_body,
        out_type=jax.ShapeDtypeStruct((TK,), jnp.int32),
        mesh=_sc_mesh(),
        compiler_params=pltpu.CompilerParams(needs_layout_passes=False),
        scratch_types=[
            pltpu.VMEM((MPW,), jnp.int32),
            pltpu.VMEM((MPW,), jnp.int32),
            pltpu.VMEM((NS, 16), jnp.int32),
            pltpu.VMEM((16,), jnp.int32),
            pltpu.VMEM((16,), jnp.int32),
        ],
    )


def _gather_body(tab_hbm, pa_hbm, pb_hbm, out_hbm,
                 sidxa0, sidxa1, sidxb0, sidxb1, rows0, rows1,
                 gsem0, gsem1, ssema0, ssema1, ssemb0, ssemb1):
    per_w = T // NW
    base = _wid() * per_w
    n_ch = per_w // GCH
    sidxa = (sidxa0, sidxa1)
    sidxb = (sidxb0, sidxb1)
    rows = (rows0, rows1)
    gsem = (gsem0, gsem1)
    ssema = (ssema0, ssema1)
    ssemb = (ssemb0, ssemb1)

    def start(c):
        b = c % 2
        lo = base + c * GCH
        pltpu.sync_copy(pa_hbm.at[pl.ds(lo, GCH)], sidxa[b])
        pltpu.sync_copy(pb_hbm.at[pl.ds(lo, GCH)], sidxb[b])
        return pltpu.async_copy(tab_hbm.at[pl.ds(lo, GCH)], rows[b], gsem[b])

    gd = {0: start(0)}
    sd = {}
    for c in range(n_ch):
        b = c % 2
        if c + 1 < n_ch:
            if c - 1 in sd:
                sd[c - 1][0].wait()
                sd[c - 1][1].wait()
            gd[c + 1] = start(c + 1)
        gd[c].wait()
        sd[c] = (pltpu.async_copy(rows[b], out_hbm.at[sidxa[b]], ssema[b]),
                 pltpu.async_copy(rows[b], out_hbm.at[sidxb[b]], ssemb[b]))
    for c in (n_ch - 2, n_ch - 1):
        if c >= 0 and c in sd:
            sd[c][0].wait()
            sd[c][1].wait()


@functools.lru_cache(maxsize=None)
def _gather_call():
    return pl.kernel(
        _gather_body,
        out_type=jax.ShapeDtypeStruct((P, HIDDEN), jnp.float32),
        mesh=_sc_mesh(),
        scratch_types=[
            pltpu.VMEM((GCH,), jnp.int32),
            pltpu.VMEM((GCH,), jnp.int32),
            pltpu.VMEM((GCH,), jnp.int32),
            pltpu.VMEM((GCH,), jnp.int32),
            pltpu.VMEM((GCH, HIDDEN), jnp.float32),
            pltpu.VMEM((GCH, HIDDEN), jnp.float32),
            pltpu.SemaphoreType.DMA,
            pltpu.SemaphoreType.DMA,
            pltpu.SemaphoreType.DMA,
            pltpu.SemaphoreType.DMA,
            pltpu.SemaphoreType.DMA,
            pltpu.SemaphoreType.DMA,
        ],
    )


def _ffn_body(be_ref, x_ref, wgu_ref, wd_ref, o_ref):
    @pl.when(pl.program_id(0) < be_ref[NB])
    def _():
        x = x_ref[...].astype(jnp.bfloat16)
        gu = lax.dot_general(
            x, wgu_ref[0].astype(jnp.bfloat16),
            (((1,), (1,)), ((), ())),
            preferred_element_type=jnp.float32)
        g = gu[:, :INTER]
        u = gu[:, INTER:]
        h = (g * lax.logistic(g) * u).astype(jnp.bfloat16)
        o_ref[...] = lax.dot_general(
            h, wd_ref[0].astype(jnp.bfloat16),
            (((1,), (1,)), ((), ())),
            preferred_element_type=jnp.float32)


def _grouped_ffn(bexp, xp, wgu, wd):
    grid_spec = pltpu.PrefetchScalarGridSpec(
        num_scalar_prefetch=1,
        grid=(NB,),
        in_specs=[
            pl.BlockSpec((BM, HIDDEN), lambda b, be: (b, 0)),
            pl.BlockSpec((1, 2 * INTER, HIDDEN), lambda b, be: (be[b], 0, 0)),
            pl.BlockSpec((1, HIDDEN, INTER), lambda b, be: (be[b], 0, 0)),
        ],
        out_specs=pl.BlockSpec((BM, HIDDEN), lambda b, be: (b, 0)),
    )
    return pl.pallas_call(
        _ffn_body,
        grid_spec=grid_spec,
        out_shape=jax.ShapeDtypeStruct((P, HIDDEN), jnp.float32),
        compiler_params=pltpu.CompilerParams(
            dimension_semantics=("arbitrary",),
            vmem_limit_bytes=110 * 1024 * 1024,
        ),
    )(bexp, xp, wgu, wd)


def _combine_body(po_hbm, pos_hbm, wts_hbm, out_hbm,
                  ii0, ii1, wv0, wv1, bi0, bi1, bo0, bo1,
                  sg0, sg1, so0, so1):
    per_w = T // NW
    tbase = _wid() * per_w
    n_ch = per_w // CT
    ii = (ii0, ii1)
    wv = (wv0, wv1)
    bi = (bi0, bi1)
    bo = (bo0, bo1)
    sg = (sg0, sg1)
    so = (so0, so1)

    def start(c):
        b = c % 2
        lo = (tbase + c * CT) * K
        pltpu.sync_copy(pos_hbm.at[pl.ds(lo, CT * K)], ii[b])
        pltpu.sync_copy(wts_hbm.at[pl.ds(lo, CT * K)], wv[b])
        return pltpu.async_copy(po_hbm.at[ii[b]], bi[b], sg[b])

    gd = {0: start(0)}
    wd_ = {}
    for c in range(n_ch):
        b = c % 2
        if c + 1 < n_ch:
            if c - 1 in wd_:
                wd_[c - 1].wait()
            gd[c + 1] = start(c + 1)
        gd[c].wait()
        wvec = wv[b][...]
        for r in range(CT):
            wa = wvec[2 * r]
            wb = wvec[2 * r + 1]

            @plsc.parallel_loop(0, HIDDEN // 16, unroll=8)
            def dbody(d, r=r, b=b, wa=wa, wb=wb):
                s = pl.ds(d * 16, 16)
                bo[b][r, s] = wa * bi[b][2 * r, s] + wb * bi[b][2 * r + 1, s]
        wd_[c] = pltpu.async_copy(
            bo[b], out_hbm.at[pl.ds(tbase + c * CT, CT)], so[b])
    for c in (n_ch - 2, n_ch - 1):
        if c >= 0 and c in wd_:
            wd_[c].wait()


@functools.lru_cache(maxsize=None)
def _combine_call():
    return pl.kernel(
        _combine_body,
        out_type=jax.ShapeDtypeStruct((T, HIDDEN), jnp.float32),
        mesh=_sc_mesh(),
        scratch_types=[
            pltpu.VMEM((CT * K,), jnp.int32),
            pltpu.VMEM((CT * K,), jnp.int32),
            pltpu.VMEM((CT * K,), jnp.float32),
            pltpu.VMEM((CT * K,), jnp.float32),
            pltpu.VMEM((CT * K, HIDDEN), jnp.float32),
            pltpu.VMEM((CT * K, HIDDEN), jnp.float32),
            pltpu.VMEM((CT, HIDDEN), jnp.float32),
            pltpu.VMEM((CT, HIDDEN), jnp.float32),
            pltpu.SemaphoreType.DMA,
            pltpu.SemaphoreType.DMA,
            pltpu.SemaphoreType.DMA,
            pltpu.SemaphoreType.DMA,
        ],
    )


def kernel(hidden_states, top_k_index, top_k_weights, gate_up_proj, down_proj):
    tki_flat = top_k_index.astype(jnp.int32).reshape(-1)
    ranks, cnts = _meta1_call()(tki_flat)
    pos = _meta2_call()(tki_flat, ranks, cnts)
    tot = jnp.sum(cnts, axis=0)[:E]
    nb = (tot + BM - 1) // BM
    bcum = jnp.cumsum(nb)
    bexp = jnp.minimum(
        jnp.searchsorted(bcum, jnp.arange(NB, dtype=jnp.int32), side="right"),
        E - 1).astype(jnp.int32)
    bmeta = jnp.concatenate([bexp, bcum[-1:].astype(jnp.int32)])

    pos2 = pos.reshape(T, K)
    packed_x = _gather_call()(hidden_states, pos2[:, 0], pos2[:, 1])

    packed_out = _grouped_ffn(bmeta, packed_x, gate_up_proj, down_proj)

    final = _combine_call()(
        packed_out, pos, top_k_weights.reshape(-1).astype(jnp.float32))
    return final
